# Initial kernel scaffold; baseline (speedup 1.0000x reference)
import jax
import jax.numpy as jnp
from jax.experimental import pallas as pl


def kernel(input, table, W, b):
    raise NotImplementedError("write your pallas kernel here")


# speedup vs baseline: 100.4475x; 100.4475x over previous
import functools

import jax
import jax.numpy as jnp
from jax import lax
from jax.experimental import pallas as pl
from jax.experimental.pallas import tpu as pltpu
from jax.experimental.pallas import tpu_sc as plsc

VOCAB = 100277
DIM = 20
DIMP = 24
B = 16384


def _sc_info():
    try:
        info = plsc.get_sparse_core_info()
        return info.num_cores, info.num_subcores
    except Exception:
        return 2, 16


def _make_gather(num_cores, num_subcores):
    nw = num_cores * num_subcores
    bpw = B // nw
    mesh = plsc.VectorSubcoreMesh(core_axis_name="c", subcore_axis_name="s")

    @functools.partial(
        pl.kernel,
        mesh=mesh,
        out_type=jax.ShapeDtypeStruct((B, DIMP), jnp.float32),
        scratch_types=[
            pltpu.VMEM((bpw,), jnp.int32),
            pltpu.VMEM((bpw, DIMP), jnp.float32),
            pltpu.SemaphoreType.DMA,
        ],
        compiler_params=pltpu.CompilerParams(use_tc_tiling_on_sc=False),
    )
    def gather(table_hbm, idx_hbm, out_hbm, idx_v, rows_v, sem):
        wid = lax.axis_index("s") * num_cores + lax.axis_index("c")
        base = wid * bpw
        pltpu.sync_copy(idx_hbm.at[pl.ds(base, bpw)], idx_v)
        pltpu.async_copy(table_hbm.at[idx_v], rows_v, sem).wait()
        pltpu.sync_copy(rows_v, out_hbm.at[pl.ds(base, bpw)])

    return gather


def _linear_body(rows_ref, w_ref, b_ref, out_ref):
    out_ref[...] = (
        lax.dot_general(
            rows_ref[...],
            w_ref[...],
            (((1,), (1,)), ((), ())),
            preferred_element_type=jnp.float32,
        )
        + b_ref[...]
    )


def kernel(input, table, W, b):
    ids = input[:, -1].astype(jnp.int32)
    table_p = jnp.pad(table, ((0, 0), (0, DIMP - DIM)))
    w_p = jnp.pad(W, ((0, 0), (0, DIMP - DIM)))
    nc, ns = _sc_info()
    rows = _make_gather(nc, ns)(table_p, ids)
    out = pl.pallas_call(
        _linear_body,
        out_shape=jax.ShapeDtypeStruct((B, DIM), jnp.float32),
    )(rows, w_p, b.reshape(1, DIM))
    return out


# speedup vs baseline: 237.5062x; 2.3645x over previous
import functools

import jax
import jax.numpy as jnp
from jax import lax
from jax.experimental import pallas as pl
from jax.experimental.pallas import tpu as pltpu
from jax.experimental.pallas import tpu_sc as plsc

VOCAB = 100277
DIM = 20
B = 16384
LANES = 16


def _sc_info():
    try:
        info = plsc.get_sparse_core_info()
        return info.num_cores, info.num_subcores
    except Exception:
        return 2, 16


def _make_gather(num_cores, num_subcores):
    nw = num_cores * num_subcores
    bpw = B // nw
    nchunk = bpw // LANES
    mesh = plsc.VectorSubcoreMesh(core_axis_name="c", subcore_axis_name="s")

    @functools.partial(
        pl.kernel,
        mesh=mesh,
        out_type=jax.ShapeDtypeStruct((DIM, B), jnp.float32),
        scratch_types=[
            pltpu.VMEM((bpw,), jnp.int32),
            pltpu.VMEM((DIM * bpw,), jnp.int32),
            pltpu.VMEM((DIM * bpw,), jnp.float32),
            pltpu.SemaphoreType.DMA,
            pltpu.SemaphoreType.DMA,
        ],
    )
    def gather(tflat_hbm, idx_hbm, out_hbm, idx_v, ilist_v, rows_v, sem, sem2):
        wid = lax.axis_index("s") * num_cores + lax.axis_index("c")
        base = wid * bpw
        pltpu.sync_copy(idx_hbm.at[pl.ds(base, bpw)], idx_v)
        for c in range(nchunk):
            v = idx_v[pl.ds(c * LANES, LANES)]
            for d in range(DIM):
                ilist_v[pl.ds(d * bpw + c * LANES, LANES)] = v + d * VOCAB
        pltpu.async_copy(tflat_hbm.at[ilist_v], rows_v, sem).wait()
        copies = [
            pltpu.async_copy(
                rows_v.at[pl.ds(d * bpw, bpw)],
                out_hbm.at[d, pl.ds(base, bpw)],
                sem2,
            )
            for d in range(DIM)
        ]
        for cp in copies:
            cp.wait()

    return gather


def _linear_body(rows_ref, w_ref, b_ref, out_ref):
    out_ref[...] = (
        lax.dot_general(
            rows_ref[...],
            w_ref[...],
            (((0,), (1,)), ((), ())),
            preferred_element_type=jnp.float32,
        )
        + b_ref[...]
    )


def kernel(input, table, W, b):
    ids = input[:, -1].astype(jnp.int32)
    tflat = table.T.reshape(-1)
    nc, ns = _sc_info()
    rows_t = _make_gather(nc, ns)(tflat, ids)
    out = pl.pallas_call(
        _linear_body,
        out_shape=jax.ShapeDtypeStruct((B, DIM), jnp.float32),
    )(rows_t, W, b.reshape(1, DIM))
    return out
